# baseline (speedup 1.0000x reference)
import functools
import math

import jax
import jax.numpy as jnp
from jax import lax
from jax.experimental import pallas as pl
from jax.experimental.pallas import tpu as pltpu
from jax.experimental.pallas import tpu_sc as plsc

NUM_HEADS = 12
NUM_BUCKETS = 32
MAX_DISTANCE = 50000
SEQ = 2048

NUM_CORES = 2
NUM_SUBCORES = 16
NW = NUM_CORES * NUM_SUBCORES
ROWS_PER_W = SEQ // NW
LANES = 16
CHUNKS = SEQ // LANES


def _bucket_lut():
    n = jnp.arange(MAX_DISTANCE, dtype=jnp.int32)
    max_exact = NUM_BUCKETS // 2
    n_large = jnp.maximum(n, max_exact).astype(jnp.float32)
    val_if_large = max_exact + (
        jnp.log(n_large / max_exact)
        / math.log(MAX_DISTANCE / max_exact)
        * (NUM_BUCKETS - max_exact - 1)
    ).astype(jnp.int32)
    val_if_large = jnp.minimum(val_if_large, NUM_BUCKETS - 1)
    return jnp.where(n < max_exact, n, val_if_large)


def _sc_bias(d2, lut, wt):
    mesh = plsc.VectorSubcoreMesh(core_axis_name="c", subcore_axis_name="s")

    @functools.partial(
        pl.kernel,
        out_type=jax.ShapeDtypeStruct((NUM_HEADS, SEQ, SEQ), jnp.float32),
        mesh=mesh,
        compiler_params=pltpu.CompilerParams(needs_layout_passes=False),
        scratch_types=[
            pltpu.VMEM((MAX_DISTANCE,), jnp.int32),
            pltpu.VMEM((NUM_HEADS * NUM_BUCKETS,), jnp.float32),
            pltpu.VMEM((2, SEQ), jnp.int32),
            pltpu.VMEM((2, NUM_HEADS, SEQ), jnp.float32),
            pltpu.SemaphoreType.DMA,
            pltpu.SemaphoreType.DMA,
            pltpu.SemaphoreType.DMA,
            pltpu.SemaphoreType.DMA,
        ],
    )
    def body(d_hbm, lut_hbm, wt_hbm, out_hbm, lut_v, wt_v, dbuf, obuf,
             sem_in0, sem_in1, sem_out0, sem_out1):
        wid = lax.axis_index("s") * NUM_CORES + lax.axis_index("c")
        sem_in = (sem_in0, sem_in1)
        sem_out = (sem_out0, sem_out1)
        pltpu.sync_copy(lut_hbm, lut_v)
        pltpu.sync_copy(wt_hbm, wt_v)
        row0 = wid * ROWS_PER_W

        pltpu.async_copy(d_hbm.at[row0], dbuf.at[0], sem_in[0])

        def do_chunk(b, c):
            base = pl.multiple_of(c * LANES, LANES)
            dvec = dbuf[b, pl.ds(base, LANES)]
            bvec = plsc.load_gather(lut_v, [dvec])
            for h in range(NUM_HEADS):
                w = plsc.load_gather(wt_v, [bvec + (h * NUM_BUCKETS)])
                obuf[b, h, pl.ds(base, LANES)] = w

        @pl.loop(0, ROWS_PER_W, step=2)
        def _rows(r):
            for b in range(2):
                row = row0 + r + b
                pltpu.make_async_copy(d_hbm.at[row], dbuf.at[b], sem_in[b]).wait()
                if b == 0:
                    pltpu.async_copy(d_hbm.at[row + 1], dbuf.at[1], sem_in[1])
                else:
                    @pl.when(r < ROWS_PER_W - 2)
                    def _():
                        pltpu.async_copy(d_hbm.at[row + 1], dbuf.at[0], sem_in[0])
                @pl.when(r >= 2)
                def _():
                    for h in range(NUM_HEADS):
                        pltpu.make_async_copy(
                            obuf.at[b, h], out_hbm.at[h, row], sem_out[b]
                        ).wait()

                if b == 0:
                    @pl.when(r >= 2)
                    def _():
                        for h in range(NUM_HEADS):
                            do_chunk(0, h)
                            pltpu.async_copy(
                                obuf.at[1, h], out_hbm.at[h, row - 1], sem_out[1]
                            )

                    @pl.when(r < 2)
                    def _():
                        for h in range(NUM_HEADS):
                            do_chunk(0, h)
                else:
                    for h in range(NUM_HEADS):
                        do_chunk(1, h)
                        pltpu.async_copy(
                            obuf.at[0, h], out_hbm.at[h, row - 1], sem_out[0]
                        )

                @plsc.parallel_loop(NUM_HEADS, CHUNKS, unroll=4)
                def _chunk(c):
                    do_chunk(b, c)

        last = row0 + ROWS_PER_W - 1
        for h in range(NUM_HEADS):
            pltpu.async_copy(obuf.at[1, h], out_hbm.at[h, last], sem_out[1])
        for h in range(NUM_HEADS):
            pltpu.make_async_copy(
                obuf.at[0, h], out_hbm.at[h, last - 1], sem_out[0]
            ).wait()
        for h in range(NUM_HEADS):
            pltpu.make_async_copy(
                obuf.at[1, h], out_hbm.at[h, last], sem_out[1]
            ).wait()

    return body(d2, lut, wt)


def kernel(distances, W):
    d2 = distances.reshape(SEQ, SEQ)
    wt = W.T.reshape(NUM_HEADS * NUM_BUCKETS)
    lut = _bucket_lut()
    out = _sc_bias(d2, lut, wt)
    return out.reshape(1, NUM_HEADS, SEQ, SEQ)


# speedup vs baseline: 1.3270x; 1.3270x over previous
import functools
import math

import jax
import jax.numpy as jnp
from jax import lax
from jax.experimental import pallas as pl
from jax.experimental.pallas import tpu as pltpu
from jax.experimental.pallas import tpu_sc as plsc

NUM_HEADS = 12
NUM_BUCKETS = 32
MAX_DISTANCE = 50000
SEQ = 2048

NUM_CORES = 2
NUM_SUBCORES = 16
NW = NUM_CORES * NUM_SUBCORES
ROWS_PER_W = SEQ // NW
LANES = 16
CHUNKS = SEQ // LANES


def _bucket_lut():
    n = jnp.arange(MAX_DISTANCE, dtype=jnp.int32)
    max_exact = NUM_BUCKETS // 2
    n_large = jnp.maximum(n, max_exact).astype(jnp.float32)
    val_if_large = max_exact + (
        jnp.log(n_large / max_exact)
        / math.log(MAX_DISTANCE / max_exact)
        * (NUM_BUCKETS - max_exact - 1)
    ).astype(jnp.int32)
    val_if_large = jnp.minimum(val_if_large, NUM_BUCKETS - 1)
    return jnp.where(n < max_exact, n, val_if_large)


def _sc_bias(d2, lut, wt):
    mesh = plsc.VectorSubcoreMesh(core_axis_name="c", subcore_axis_name="s")

    @functools.partial(
        pl.kernel,
        out_type=jax.ShapeDtypeStruct((NUM_HEADS, SEQ, SEQ), jnp.float32),
        mesh=mesh,
        compiler_params=pltpu.CompilerParams(needs_layout_passes=False),
        scratch_types=[
            pltpu.VMEM((MAX_DISTANCE,), jnp.int32),
            pltpu.VMEM((NUM_HEADS * NUM_BUCKETS,), jnp.float32),
            pltpu.VMEM((2, SEQ), jnp.int32),
            pltpu.VMEM((2, NUM_HEADS, SEQ), jnp.float32),
            pltpu.SemaphoreType.DMA,
            pltpu.SemaphoreType.DMA,
            pltpu.SemaphoreType.DMA,
            pltpu.SemaphoreType.DMA,
        ],
    )
    def body(d_hbm, lut_hbm, wt_hbm, out_hbm, lut_v, wt_v, dbuf, obuf,
             sem_in0, sem_in1, sem_out0, sem_out1):
        wid = lax.axis_index("s") * NUM_CORES + lax.axis_index("c")
        sem_in = (sem_in0, sem_in1)
        sem_out = (sem_out0, sem_out1)
        pltpu.sync_copy(lut_hbm, lut_v)
        pltpu.sync_copy(wt_hbm, wt_v)
        row0 = wid * ROWS_PER_W

        pltpu.async_copy(d_hbm.at[row0], dbuf.at[0], sem_in[0])

        @pl.loop(0, ROWS_PER_W, step=2)
        def _rows(r):
            for b in range(2):
                row = row0 + r + b
                pltpu.make_async_copy(d_hbm.at[row], dbuf.at[b], sem_in[b]).wait()
                if b == 0:
                    pltpu.async_copy(d_hbm.at[row + 1], dbuf.at[1], sem_in[1])
                else:
                    @pl.when(r < ROWS_PER_W - 2)
                    def _():
                        pltpu.async_copy(d_hbm.at[row + 1], dbuf.at[0], sem_in[0])
                @pl.when(r >= 2)
                def _():
                    for h in range(NUM_HEADS):
                        pltpu.make_async_copy(
                            obuf.at[b, h], out_hbm.at[h, row], sem_out[b]
                        ).wait()

                @plsc.parallel_loop(0, CHUNKS, unroll=4)
                def _chunk(c):
                    base = pl.multiple_of(c * LANES, LANES)
                    dvec = dbuf[b, pl.ds(base, LANES)]
                    bvec = plsc.load_gather(lut_v, [dvec])
                    for h in range(NUM_HEADS):
                        w = plsc.load_gather(wt_v, [bvec + (h * NUM_BUCKETS)])
                        obuf[b, h, pl.ds(base, LANES)] = w

                for h in range(NUM_HEADS):
                    pltpu.async_copy(obuf.at[b, h], out_hbm.at[h, row], sem_out[b])

        for b in range(2):
            row = row0 + ROWS_PER_W - 2 + b
            for h in range(NUM_HEADS):
                pltpu.make_async_copy(
                    obuf.at[b, h], out_hbm.at[h, row], sem_out[b]
                ).wait()

    return body(d2, lut, wt)


def kernel(distances, W):
    d2 = distances.reshape(SEQ, SEQ)
    wt = W.T.reshape(NUM_HEADS * NUM_BUCKETS)
    lut = _bucket_lut()
    out = _sc_bias(d2, lut, wt)
    return out.reshape(1, NUM_HEADS, SEQ, SEQ)
